# baseline (speedup 1.0000x reference)
import jax
import jax.numpy as jnp
from jax.experimental import pallas as pl

_S = 30
_EPS = 1e-3
_C = 2 * _S + 1
_BN = 65536
_BNL = _BN // 8


def _two_hot_body(x_ref, out_ref):
    x = x_ref[...]
    h = jnp.sign(x) * (jnp.sqrt(jnp.abs(x) + 1.0) - 1.0) + _EPS * x
    t = jnp.clip(h, -float(_S), float(_S)) + float(_S)
    t = t.reshape(1, _BN)
    col = jax.lax.broadcasted_iota(jnp.int32, (_C, 1), 0).astype(jnp.float32)
    out_ref[...] = jnp.maximum(1.0 - jnp.abs(t - col), 0.0)


def kernel(x):
    n = x.shape[0]
    g = n // _BN
    xg = x.reshape(g * 8, _BNL)
    out_t = pl.pallas_call(
        _two_hot_body,
        grid=(g,),
        in_specs=[pl.BlockSpec((8, _BNL), lambda j: (j, 0))],
        out_specs=pl.BlockSpec((_C, _BN), lambda j: (0, j)),
        out_shape=jax.ShapeDtypeStruct((_C, n), jnp.float32),
    )(xg)
    return out_t.T
